# Initial kernel scaffold; baseline (speedup 1.0000x reference)
import jax
import jax.numpy as jnp
from jax.experimental import pallas as pl


def kernel(mem, last_update, t, edge_feat, time_w, time_b, W_ih, W_hh, b_ih, b_hh, src, dst):
    raise NotImplementedError("write your pallas kernel here")


# speedup vs baseline: 1.8624x; 1.8624x over previous
import jax
import jax.numpy as jnp
from jax import lax
from jax.experimental import pallas as pl
from jax.experimental.pallas import tpu as pltpu

N = 100000
D = 128
B = 16384
DE = 16
TD = 16
MSG = 2 * D + DE + TD

_BLK = 512


def _gru_body(sums_ref, cnt_ref, wih_ref, whh_ref, bih_ref, bhh_ref, out_ref):
    cnt = cnt_ref[:]
    recip = 1.0 / jnp.maximum(cnt, 1.0)
    old = sums_ref[:, :D]
    rest = sums_ref[:, D:] * recip[:, None]
    aggb = jnp.concatenate([old, rest], axis=1)
    gi = lax.dot_general(aggb, wih_ref[:], (((1,), (0,)), ((), ())),
                         preferred_element_type=jnp.float32) + bih_ref[:][None, :]
    gh = lax.dot_general(old, whh_ref[:], (((1,), (0,)), ((), ())),
                         preferred_element_type=jnp.float32) + bhh_ref[:][None, :]
    i_r, i_z, i_n = gi[:, :D], gi[:, D:2 * D], gi[:, 2 * D:]
    h_r, h_z, h_n = gh[:, :D], gh[:, D:2 * D], gh[:, 2 * D:]
    r = jax.nn.sigmoid(i_r + h_r)
    z = jax.nn.sigmoid(i_z + h_z)
    n = jnp.tanh(i_n + r * h_n)
    out_ref[:] = (1.0 - z) * n + z * old


def _gru(sums, cnt, wih_t, whh_t, b_ih, b_hh):
    grid = (B // _BLK,)
    return pl.pallas_call(
        _gru_body,
        grid=grid,
        in_specs=[
            pl.BlockSpec((_BLK, MSG), lambda i: (i, 0)),
            pl.BlockSpec((_BLK,), lambda i: (i,)),
            pl.BlockSpec((MSG, 3 * D), lambda i: (0, 0)),
            pl.BlockSpec((D, 3 * D), lambda i: (0, 0)),
            pl.BlockSpec((3 * D,), lambda i: (0,)),
            pl.BlockSpec((3 * D,), lambda i: (0,)),
        ],
        out_specs=pl.BlockSpec((_BLK, D), lambda i: (i, 0)),
        out_shape=jax.ShapeDtypeStruct((B, D), jnp.float32),
    )(sums, cnt, wih_t, whh_t, b_ih, b_hh)


def kernel(mem, last_update, t, edge_feat, time_w, time_b, W_ih, W_hh, b_ih, b_hh, src, dst):
    src = src.astype(jnp.int32)
    dst = dst.astype(jnp.int32)
    dt = t - last_update[src]
    te = jnp.cos(dt[:, None] * time_w[None, :] + time_b[None, :])
    u = jnp.concatenate([mem[dst], edge_feat, te], axis=1)
    usum = jax.ops.segment_sum(u, src, num_segments=N)
    cnt_n = jax.ops.segment_sum(jnp.ones((B,), jnp.float32), src, num_segments=N)
    sums = jnp.concatenate([mem[src], usum[src]], axis=1)
    cnt = cnt_n[src]
    new = _gru(sums, cnt, W_ih.T, W_hh.T, b_ih, b_hh)
    return mem.at[src].set(new)
